# baseline (speedup 1.0000x reference)
import jax
import jax.numpy as jnp
from jax import lax
from jax.experimental import pallas as pl
from jax.experimental.pallas import tpu as pltpu
from jax.experimental.pallas import tpu_sc as plsc

VOCAB = 400000
D = 100
DP = 16
HID = 10
NCLS = 3
B = 4096
L = 200

NC = 2
NS = 16
NW = NC * NS
BPW = B // NW
CH = 20
NCHUNK = L // CH

PROJ_BLK = 20000


def _proj_body(emb_ref, w_ref, out_ref):
    out_ref[...] = jnp.dot(emb_ref[...], w_ref[...],
                           preferred_element_type=jnp.float32)


def _project(embed_weight, w1p):
    return pl.pallas_call(
        _proj_body,
        grid=(VOCAB // PROJ_BLK,),
        in_specs=[
            pl.BlockSpec((PROJ_BLK, D), lambda i: (i, 0)),
            pl.BlockSpec((D, DP), lambda i: (0, 0)),
        ],
        out_specs=pl.BlockSpec((PROJ_BLK, DP), lambda i: (i, 0)),
        out_shape=jax.ShapeDtypeStruct((VOCAB, DP), jnp.float32),
    )(embed_weight, w1p)


def _tr_body(i_ref, o_ref):
    o_ref[...] = i_ref[...].T[None]


def _transpose_idx(idx):
    return pl.pallas_call(
        _tr_body,
        grid=(NW,),
        in_specs=[pl.BlockSpec((BPW, L), lambda w: (w, 0))],
        out_specs=pl.BlockSpec((1, L, BPW), lambda w: (w, 0, 0)),
        out_shape=jax.ShapeDtypeStruct((NW, L, BPW), jnp.int32),
    )(idx)


def _sc_body(p_hbm, idx_hbm, out_hbm, idx_v, rows_v, acc_v, sem):
    wid = lax.axis_index("c") * NS + lax.axis_index("s")

    pltpu.sync_copy(idx_hbm.at[wid], idx_v)

    zero = jnp.zeros((DP,), jnp.float32)

    def z_body(i, c):
        acc_v[i] = zero
        return c
    lax.fori_loop(0, BPW, z_body, 0)

    def chunk_body(g, c):
        base = g * CH
        copies = [
            pltpu.async_copy(p_hbm.at[idx_v.at[base + j]], rows_v.at[j], sem)
            for j in range(CH)
        ]
        for cp in copies:
            cp.wait()

        def item_body(i, cc):
            v = rows_v[0, i]
            for j in range(1, CH):
                v = v + rows_v[j, i]
            acc_v[i] = acc_v[i] + v
            return cc
        lax.fori_loop(0, BPW, item_body, 0)
        return c
    lax.fori_loop(0, NCHUNK, chunk_body, 0)

    pltpu.sync_copy(acc_v, out_hbm.at[wid])


_sc_call = pl.kernel(
    _sc_body,
    out_type=jax.ShapeDtypeStruct((NW, BPW, DP), jnp.float32),
    mesh=plsc.VectorSubcoreMesh(core_axis_name="c", subcore_axis_name="s",
                                num_cores=NC, num_subcores=NS),
    scratch_types=[
        pltpu.VMEM((L, BPW), jnp.int32),
        pltpu.VMEM((CH, BPW, DP), jnp.float32),
        pltpu.VMEM((BPW, DP), jnp.float32),
        pltpu.SemaphoreType.DMA,
    ],
    compiler_params=pltpu.CompilerParams(use_tc_tiling_on_sc=False),
)


def _mlp_body(s_ref, b1_ref, w2_ref, b2_ref, out_ref):
    h = jnp.maximum(s_ref[...] * jnp.float32(1.0 / L) + b1_ref[...], 0.0)
    out_ref[...] = jnp.dot(h, w2_ref[...],
                           preferred_element_type=jnp.float32) + b2_ref[...]


def _mlp(sums, b1p, w2p, b2p):
    return pl.pallas_call(
        _mlp_body,
        out_shape=jax.ShapeDtypeStruct((B, DP), jnp.float32),
    )(sums, b1p, w2p, b2p)


@jax.jit
def kernel(inputs, embed_weight, W1, b1, W2, b2):
    idx = inputs.astype(jnp.int32)
    w1p = jnp.zeros((D, DP), jnp.float32).at[:, :HID].set(W1.T)
    proj = _project(embed_weight, w1p)

    idx3 = _transpose_idx(idx)
    sums = _sc_call(proj, idx3)

    b1p = jnp.zeros((1, DP), jnp.float32).at[0, :HID].set(b1)
    w2p = jnp.zeros((DP, DP), jnp.float32).at[:HID, :NCLS].set(W2.T)
    b2p = jnp.zeros((1, DP), jnp.float32).at[0, :NCLS].set(b2)

    out = _mlp(sums.reshape(B, DP), b1p, w2p, b2p)
    return out[:, :NCLS]


# speedup vs baseline: 2.5474x; 2.5474x over previous
import jax
import jax.numpy as jnp
from jax import lax
from jax.experimental import pallas as pl
from jax.experimental.pallas import tpu as pltpu
from jax.experimental.pallas import tpu_sc as plsc

VOCAB = 400000
D = 100
DP = 16
HID = 10
NCLS = 3
B = 4096
L = 200

NC = 2
NS = 16
NW = NC * NS
BPW = B // NW
CH = 20
NCHUNK = L // CH

PROJ_BLK = 16000


def _proj_body(embt_ref, w_ref, out_ref):
    rows = lax.dot_general(embt_ref[...], w_ref[...],
                           (((0,), (0,)), ((), ())),
                           preferred_element_type=jnp.float32)
    r3 = rows.reshape(PROJ_BLK // 8, 8, DP)
    out_ref[...] = jnp.concatenate([r3[:, t, :] for t in range(8)], axis=1)


def _project(embt, w1p):
    return pl.pallas_call(
        _proj_body,
        grid=(VOCAB // PROJ_BLK,),
        in_specs=[
            pl.BlockSpec((D, PROJ_BLK), lambda i: (0, i)),
            pl.BlockSpec((D, DP), lambda i: (0, 0)),
        ],
        out_specs=pl.BlockSpec((PROJ_BLK // 8, 128), lambda i: (i, 0)),
        out_shape=jax.ShapeDtypeStruct((VOCAB // 8, 128), jnp.float32),
    )(embt, w1p)


def _sc_body(p_hbm, idxt_hbm, out_hbm, idx_v, rows_v, acc_v, sem):
    wid = lax.axis_index("c") * NS + lax.axis_index("s")

    pltpu.sync_copy(idxt_hbm.at[:, pl.ds(wid * BPW, BPW)], idx_v)

    zero = jnp.zeros((DP,), jnp.float32)

    def z_body(i, c):
        acc_v[i] = zero
        return c
    lax.fori_loop(0, BPW, z_body, 0)

    def chunk_body(g, c):
        base = g * CH
        copies = [
            pltpu.async_copy(p_hbm.at[idx_v.at[base + j]], rows_v.at[j], sem)
            for j in range(CH)
        ]
        for cp in copies:
            cp.wait()

        def item_body(i, cc):
            v = rows_v[0, i]
            for j in range(1, CH):
                v = v + rows_v[j, i]
            acc_v[i] = acc_v[i] + v
            return cc
        lax.fori_loop(0, BPW, item_body, 0)
        return c
    lax.fori_loop(0, NCHUNK, chunk_body, 0)

    pltpu.sync_copy(acc_v, out_hbm.at[wid])


_sc_call = pl.kernel(
    _sc_body,
    out_type=jax.ShapeDtypeStruct((NW, BPW, DP), jnp.float32),
    mesh=plsc.VectorSubcoreMesh(core_axis_name="c", subcore_axis_name="s",
                                num_cores=NC, num_subcores=NS),
    scratch_types=[
        pltpu.VMEM((L, BPW), jnp.int32),
        pltpu.VMEM((CH, BPW, DP), jnp.float32),
        pltpu.VMEM((BPW, DP), jnp.float32),
        pltpu.SemaphoreType.DMA,
    ],
    compiler_params=pltpu.CompilerParams(use_tc_tiling_on_sc=False),
)


def _mlp_body(s_ref, b1_ref, w2_ref, b2_ref, out_ref):
    h = jnp.maximum(s_ref[...] * jnp.float32(1.0 / L) + b1_ref[...], 0.0)
    out_ref[...] = jnp.dot(h, w2_ref[...],
                           preferred_element_type=jnp.float32) + b2_ref[...]


def _mlp(sums, b1p, w2p, b2p):
    return pl.pallas_call(
        _mlp_body,
        out_shape=jax.ShapeDtypeStruct((B, DP), jnp.float32),
    )(sums, b1p, w2p, b2p)


@jax.jit
def kernel(inputs, embed_weight, W1, b1, W2, b2):
    idxt = inputs.astype(jnp.int32).T
    embt = embed_weight.T
    w1p = jnp.zeros((D, DP), jnp.float32).at[:, :HID].set(W1.T)
    proj = _project(embt, w1p)

    sums = _sc_call(proj.reshape(VOCAB, DP), idxt)

    b1p = jnp.zeros((1, DP), jnp.float32).at[0, :HID].set(b1)
    w2p = jnp.zeros((DP, DP), jnp.float32).at[:HID, :NCLS].set(W2.T)
    b2p = jnp.zeros((1, DP), jnp.float32).at[0, :NCLS].set(b2)

    out = _mlp(sums.reshape(B, DP), b1p, w2p, b2p)
    return out[:, :NCLS]


# speedup vs baseline: 2.6332x; 1.0337x over previous
import jax
import jax.numpy as jnp
from jax import lax
from jax.experimental import pallas as pl
from jax.experimental.pallas import tpu as pltpu
from jax.experimental.pallas import tpu_sc as plsc

VOCAB = 400000
D = 100
DP = 16
HID = 10
NCLS = 3
B = 4096
L = 200

NC = 2
NS = 16
NW = NC * NS
BPW = B // NW
CH = 10
NSUPER = L // (2 * CH)

PROJ_BLK = 16000


def _proj_body(embt_ref, w_ref, out_ref):
    rows = lax.dot_general(embt_ref[...], w_ref[...],
                           (((0,), (0,)), ((), ())),
                           preferred_element_type=jnp.float32)
    r3 = rows.reshape(PROJ_BLK // 8, 8, DP)
    out_ref[...] = jnp.concatenate([r3[:, t, :] for t in range(8)], axis=1)


def _project(embt, w1p):
    return pl.pallas_call(
        _proj_body,
        grid=(VOCAB // PROJ_BLK,),
        in_specs=[
            pl.BlockSpec((D, PROJ_BLK), lambda i: (0, i)),
            pl.BlockSpec((D, DP), lambda i: (0, 0)),
        ],
        out_specs=pl.BlockSpec((PROJ_BLK // 8, 128), lambda i: (i, 0)),
        out_shape=jax.ShapeDtypeStruct((VOCAB // 8, 128), jnp.float32),
    )(embt, w1p)


def _sc_body(p_hbm, idxt_hbm, out_hbm, idx_v, rows_v, acc_v, semA, semB):
    wid = lax.axis_index("c") * NS + lax.axis_index("s")

    pltpu.sync_copy(idxt_hbm.at[:, pl.ds(wid * BPW, BPW)], idx_v)

    zero = jnp.zeros((DP,), jnp.float32)

    def z_body(i, c):
        acc_v[i] = zero
        return c
    lax.fori_loop(0, BPW, z_body, 0)

    def fire(buf, base, sem):
        return [
            pltpu.async_copy(p_hbm.at[idx_v.at[base + j]],
                             rows_v.at[buf, j], sem)
            for j in range(CH)
        ]

    def drain(buf, base, sem):
        for j in range(CH):
            pltpu.make_async_copy(p_hbm.at[idx_v.at[base + j]],
                                  rows_v.at[buf, j], sem).wait()

    def reduce(buf):
        def item_body(i, cc):
            v = rows_v[buf, 0, i]
            for j in range(1, CH):
                v = v + rows_v[buf, j, i]
            acc_v[i] = acc_v[i] + v
            return cc
        lax.fori_loop(0, BPW, item_body, 0)

    fire(0, 0, semA)

    def super_body(g, c):
        base = 2 * g * CH
        drain(0, base, semA)
        fire(1, base + CH, semB)
        reduce(0)
        drain(1, base + CH, semB)

        @pl.when(g < NSUPER - 1)
        def _():
            fire(0, base + 2 * CH, semA)
        reduce(1)
        return c
    lax.fori_loop(0, NSUPER, super_body, 0)

    pltpu.sync_copy(acc_v, out_hbm.at[wid])


_sc_call = pl.kernel(
    _sc_body,
    out_type=jax.ShapeDtypeStruct((NW, BPW, DP), jnp.float32),
    mesh=plsc.VectorSubcoreMesh(core_axis_name="c", subcore_axis_name="s",
                                num_cores=NC, num_subcores=NS),
    scratch_types=[
        pltpu.VMEM((L, BPW), jnp.int32),
        pltpu.VMEM((2, CH, BPW, DP), jnp.float32),
        pltpu.VMEM((BPW, DP), jnp.float32),
        pltpu.SemaphoreType.DMA,
        pltpu.SemaphoreType.DMA,
    ],
    compiler_params=pltpu.CompilerParams(use_tc_tiling_on_sc=False),
)


def _mlp_body(s_ref, b1_ref, w2_ref, b2_ref, out_ref):
    h = jnp.maximum(s_ref[...] * jnp.float32(1.0 / L) + b1_ref[...], 0.0)
    out_ref[...] = jnp.dot(h, w2_ref[...],
                           preferred_element_type=jnp.float32) + b2_ref[...]


def _mlp(sums, b1p, w2p, b2p):
    return pl.pallas_call(
        _mlp_body,
        out_shape=jax.ShapeDtypeStruct((B, DP), jnp.float32),
    )(sums, b1p, w2p, b2p)


@jax.jit
def kernel(inputs, embed_weight, W1, b1, W2, b2):
    idxt = inputs.astype(jnp.int32).T
    embt = embed_weight.T
    w1p = jnp.zeros((D, DP), jnp.float32).at[:, :HID].set(W1.T)
    proj = _project(embt, w1p)

    sums = _sc_call(proj.reshape(VOCAB, DP), idxt)

    b1p = jnp.zeros((1, DP), jnp.float32).at[0, :HID].set(b1)
    w2p = jnp.zeros((DP, DP), jnp.float32).at[:HID, :NCLS].set(W2.T)
    b2p = jnp.zeros((1, DP), jnp.float32).at[0, :NCLS].set(b2)

    out = _mlp(sums.reshape(B, DP), b1p, w2p, b2p)
    return out[:, :NCLS]


# speedup vs baseline: 2.6930x; 1.0227x over previous
import jax
import jax.numpy as jnp
from jax import lax
from jax.experimental import pallas as pl
from jax.experimental.pallas import tpu as pltpu
from jax.experimental.pallas import tpu_sc as plsc

VOCAB = 400000
D = 100
DP = 16
HID = 10
NCLS = 3
B = 4096
L = 200

NC = 2
NS = 16
NW = NC * NS
BPW = B // NW
CH = 10
NSUPER = L // (2 * CH)

PROJ_BLK = 16000


def _proj_body(embt_ref, w_ref, out_ref):
    rows = lax.dot_general(embt_ref[...], w_ref[...],
                           (((0,), (0,)), ((), ())),
                           preferred_element_type=jnp.float32)
    r3 = rows.reshape(PROJ_BLK // 8, 8, DP)
    out_ref[...] = jnp.concatenate([r3[:, t, :] for t in range(8)], axis=1)


def _project(embt, w1p):
    return pl.pallas_call(
        _proj_body,
        grid=(VOCAB // PROJ_BLK,),
        in_specs=[
            pl.BlockSpec((D, PROJ_BLK), lambda i: (0, i)),
            pl.BlockSpec((D, DP), lambda i: (0, 0)),
        ],
        out_specs=pl.BlockSpec((PROJ_BLK // 8, 128), lambda i: (i, 0)),
        out_shape=jax.ShapeDtypeStruct((VOCAB // 8, 128), jnp.float32),
    )(embt, w1p)


def _sc_body(p_hbm, idxt_hbm, out_hbm, idx_v, rows_v, acc_v, semA, semB):
    wid = lax.axis_index("c") * NS + lax.axis_index("s")

    pltpu.sync_copy(idxt_hbm.at[:, wid], idx_v)

    zero = jnp.zeros((DP,), jnp.float32)

    def z_body(i, c):
        acc_v[i] = zero
        return c
    lax.fori_loop(0, BPW, z_body, 0)

    def fire(buf, base, sem):
        return [
            pltpu.async_copy(p_hbm.at[idx_v.at[(base + j) // 8, (base + j) % 8]],
                             rows_v.at[buf, j], sem)
            for j in range(CH)
        ]

    def drain(buf, base, sem):
        for j in range(CH):
            pltpu.make_async_copy(
                p_hbm.at[idx_v.at[(base + j) // 8, (base + j) % 8]],
                rows_v.at[buf, j], sem).wait()

    def reduce(buf):
        def item_body(i, cc):
            v = rows_v[buf, 0, i]
            for j in range(1, CH):
                v = v + rows_v[buf, j, i]
            acc_v[i] = acc_v[i] + v
            return cc
        lax.fori_loop(0, BPW, item_body, 0)

    fire(0, 0, semA)

    def super_body(g, c):
        base = 2 * g * CH
        drain(0, base, semA)
        fire(1, base + CH, semB)
        reduce(0)
        drain(1, base + CH, semB)

        @pl.when(g < NSUPER - 1)
        def _():
            fire(0, base + 2 * CH, semA)
        reduce(1)
        return c
    lax.fori_loop(0, NSUPER, super_body, 0)

    pltpu.sync_copy(acc_v, out_hbm.at[wid])


_sc_call = pl.kernel(
    _sc_body,
    out_type=jax.ShapeDtypeStruct((NW, BPW, DP), jnp.float32),
    mesh=plsc.VectorSubcoreMesh(core_axis_name="c", subcore_axis_name="s",
                                num_cores=NC, num_subcores=NS),
    scratch_types=[
        pltpu.VMEM((L // 8, 8, BPW), jnp.int32),
        pltpu.VMEM((2, CH, BPW, DP), jnp.float32),
        pltpu.VMEM((BPW, DP), jnp.float32),
        pltpu.SemaphoreType.DMA,
        pltpu.SemaphoreType.DMA,
    ],
    compiler_params=pltpu.CompilerParams(use_tc_tiling_on_sc=False),
)


def _mlp_body(s_ref, b1_ref, w2_ref, b2_ref, out_ref):
    h = jnp.maximum(s_ref[...] * jnp.float32(1.0 / L) + b1_ref[...], 0.0)
    out_ref[...] = jnp.dot(h, w2_ref[...],
                           preferred_element_type=jnp.float32) + b2_ref[...]


def _mlp(sums, b1p, w2p, b2p):
    return pl.pallas_call(
        _mlp_body,
        out_shape=jax.ShapeDtypeStruct((B, DP), jnp.float32),
    )(sums, b1p, w2p, b2p)


@jax.jit
def kernel(inputs, embed_weight, W1, b1, W2, b2):
    idxt = (inputs.astype(jnp.int32).T
            .reshape(L // 8, 8, NW, BPW).transpose(0, 2, 1, 3))
    embt = embed_weight.T
    w1p = jnp.zeros((D, DP), jnp.float32).at[:, :HID].set(W1.T)
    proj = _project(embt, w1p)

    sums = _sc_call(proj.reshape(VOCAB, DP), idxt)

    b1p = jnp.zeros((1, DP), jnp.float32).at[0, :HID].set(b1)
    w2p = jnp.zeros((DP, DP), jnp.float32).at[:HID, :NCLS].set(W2.T)
    b2p = jnp.zeros((1, DP), jnp.float32).at[0, :NCLS].set(b2)

    out = _mlp(sums.reshape(B, DP), b1p, w2p, b2p)
    return out[:, :NCLS]


# speedup vs baseline: 2.6962x; 1.0012x over previous
import jax
import jax.numpy as jnp
from jax import lax
from jax.experimental import pallas as pl
from jax.experimental.pallas import tpu as pltpu
from jax.experimental.pallas import tpu_sc as plsc

VOCAB = 400000
D = 100
DP = 16
HID = 10
NCLS = 3
B = 4096
L = 200

NC = 2
NS = 16
NW = NC * NS
BPW = B // NW
CH = 10
NSUPER = L // (2 * CH)

PROJ_BLK = 16000


def _proj_body(embt_ref, w_ref, out_ref):
    rows = lax.dot_general(embt_ref[...], w_ref[...],
                           (((0,), (0,)), ((), ())),
                           preferred_element_type=jnp.float32)
    r3 = rows.reshape(PROJ_BLK // 8, 8, DP)
    out_ref[...] = jnp.concatenate([r3[:, t, :] for t in range(8)], axis=1)


def _project(embt, w1p):
    return pl.pallas_call(
        _proj_body,
        grid=(VOCAB // PROJ_BLK,),
        in_specs=[
            pl.BlockSpec((D, PROJ_BLK), lambda i: (0, i)),
            pl.BlockSpec((D, DP), lambda i: (0, 0)),
        ],
        out_specs=pl.BlockSpec((PROJ_BLK // 8, 128), lambda i: (i, 0)),
        out_shape=jax.ShapeDtypeStruct((VOCAB // 8, 128), jnp.float32),
    )(embt, w1p)


def _sc_body(p_hbm, idxt_hbm, consts_hbm, out_hbm,
             idx_v, rows_v, acc_v, consts_v, semA, semB):
    wid = lax.axis_index("c") * NS + lax.axis_index("s")

    pltpu.sync_copy(idxt_hbm.at[:, wid], idx_v)
    pltpu.sync_copy(consts_hbm, consts_v)

    zero = jnp.zeros((DP,), jnp.float32)

    def z_body(i, c):
        acc_v[i] = zero
        return c
    lax.fori_loop(0, BPW, z_body, 0)

    def fire(buf, base, sem):
        return [
            pltpu.async_copy(p_hbm.at[idx_v.at[(base + j) // 8, (base + j) % 8]],
                             rows_v.at[buf, j], sem)
            for j in range(CH)
        ]

    def drain(buf, base, sem):
        for j in range(CH):
            pltpu.make_async_copy(
                p_hbm.at[idx_v.at[(base + j) // 8, (base + j) % 8]],
                rows_v.at[buf, j], sem).wait()

    def reduce(buf):
        def item_body(i, cc):
            v = rows_v[buf, 0, i]
            for j in range(1, CH):
                v = v + rows_v[buf, j, i]
            acc_v[i] = acc_v[i] + v
            return cc
        lax.fori_loop(0, BPW, item_body, 0)

    fire(0, 0, semA)

    def super_body(g, c):
        base = 2 * g * CH
        drain(0, base, semA)
        fire(1, base + CH, semB)
        reduce(0)
        drain(1, base + CH, semB)

        @pl.when(g < NSUPER - 1)
        def _():
            fire(0, base + 2 * CH, semA)
        reduce(1)
        return c
    lax.fori_loop(0, NSUPER, super_body, 0)

    b1v = consts_v[0]
    b2v = consts_v[1]
    inv = jnp.float32(1.0 / L)

    def mlp_body(i, c):
        h = jnp.maximum(acc_v[i] * inv + b1v, jnp.float32(0.0))
        o = b2v
        for j in range(HID):
            o = o + h[j] * consts_v[2 + j]
        acc_v[i] = o
        return c
    lax.fori_loop(0, BPW, mlp_body, 0)

    pltpu.sync_copy(acc_v, out_hbm.at[wid])


_sc_call = pl.kernel(
    _sc_body,
    out_type=jax.ShapeDtypeStruct((NW, BPW, DP), jnp.float32),
    mesh=plsc.VectorSubcoreMesh(core_axis_name="c", subcore_axis_name="s",
                                num_cores=NC, num_subcores=NS),
    scratch_types=[
        pltpu.VMEM((L // 8, 8, BPW), jnp.int32),
        pltpu.VMEM((2, CH, BPW, DP), jnp.float32),
        pltpu.VMEM((BPW, DP), jnp.float32),
        pltpu.VMEM((2 + HID, DP), jnp.float32),
        pltpu.SemaphoreType.DMA,
        pltpu.SemaphoreType.DMA,
    ],
    compiler_params=pltpu.CompilerParams(use_tc_tiling_on_sc=False),
)


@jax.jit
def kernel(inputs, embed_weight, W1, b1, W2, b2):
    idxt = (inputs.astype(jnp.int32).T
            .reshape(L // 8, 8, NW, BPW).transpose(0, 2, 1, 3))
    embt = embed_weight.T
    w1p = jnp.zeros((D, DP), jnp.float32).at[:, :HID].set(W1.T)
    proj = _project(embt, w1p)

    consts = jnp.zeros((2 + HID, DP), jnp.float32)
    consts = consts.at[0, :HID].set(b1)
    consts = consts.at[1, :NCLS].set(b2)
    consts = consts.at[2:2 + HID, :NCLS].set(W2.T)

    out = _sc_call(proj.reshape(VOCAB, DP), idxt, consts)
    return out.reshape(B, DP)[:, :NCLS]


# speedup vs baseline: 2.7658x; 1.0258x over previous
import jax
import jax.numpy as jnp
from jax import lax
from jax.experimental import pallas as pl
from jax.experimental.pallas import tpu as pltpu
from jax.experimental.pallas import tpu_sc as plsc

VOCAB = 400000
D = 100
DP = 16
HID = 10
NCLS = 3
B = 4096
L = 200

NC = 2
NS = 16
NW = NC * NS
BPW = B // NW
CH = 20
NSUPER = L // (2 * CH)

PROJ_BLK = 16000


def _proj_body(embt_ref, w_ref, out_ref):
    rows = lax.dot_general(embt_ref[...], w_ref[...],
                           (((0,), (0,)), ((), ())),
                           preferred_element_type=jnp.float32)
    r3 = rows.reshape(PROJ_BLK // 8, 8, DP)
    out_ref[...] = jnp.concatenate([r3[:, t, :] for t in range(8)], axis=1)


def _project(embt, w1p):
    return pl.pallas_call(
        _proj_body,
        grid=(VOCAB // PROJ_BLK,),
        in_specs=[
            pl.BlockSpec((D, PROJ_BLK), lambda i: (0, i)),
            pl.BlockSpec((D, DP), lambda i: (0, 0)),
        ],
        out_specs=pl.BlockSpec((PROJ_BLK // 8, 128), lambda i: (i, 0)),
        out_shape=jax.ShapeDtypeStruct((VOCAB // 8, 128), jnp.float32),
    )(embt, w1p)


def _sc_body(p_hbm, idxt_hbm, consts_hbm, out_hbm,
             idx_v, rows_v, acc_v, consts_v, semA, semB):
    wid = lax.axis_index("c") * NS + lax.axis_index("s")

    pltpu.sync_copy(idxt_hbm.at[:, wid], idx_v)
    pltpu.sync_copy(consts_hbm, consts_v)

    zero = jnp.zeros((DP,), jnp.float32)

    def z_body(i, c):
        acc_v[i] = zero
        return c
    lax.fori_loop(0, BPW, z_body, 0)

    def fire(buf, base, sem):
        return [
            pltpu.async_copy(p_hbm.at[idx_v.at[(base + j) // 8, (base + j) % 8]],
                             rows_v.at[buf, j], sem)
            for j in range(CH)
        ]

    def drain(buf, base, sem):
        for j in range(CH):
            pltpu.make_async_copy(
                p_hbm.at[idx_v.at[(base + j) // 8, (base + j) % 8]],
                rows_v.at[buf, j], sem).wait()

    def reduce(buf):
        def item_body(i, cc):
            v = rows_v[buf, 0, i]
            for j in range(1, CH):
                v = v + rows_v[buf, j, i]
            acc_v[i] = acc_v[i] + v
            return cc
        lax.fori_loop(0, BPW, item_body, 0)

    fire(0, 0, semA)

    def super_body(g, c):
        base = 2 * g * CH
        drain(0, base, semA)
        fire(1, base + CH, semB)
        reduce(0)
        drain(1, base + CH, semB)

        @pl.when(g < NSUPER - 1)
        def _():
            fire(0, base + 2 * CH, semA)
        reduce(1)
        return c
    lax.fori_loop(0, NSUPER, super_body, 0)

    b1v = consts_v[0]
    b2v = consts_v[1]
    inv = jnp.float32(1.0 / L)

    def mlp_body(i, c):
        h = jnp.maximum(acc_v[i] * inv + b1v, jnp.float32(0.0))
        o = b2v
        for j in range(HID):
            o = o + h[j] * consts_v[2 + j]
        acc_v[i] = o
        return c
    lax.fori_loop(0, BPW, mlp_body, 0)

    pltpu.sync_copy(acc_v, out_hbm.at[wid])


_sc_call = pl.kernel(
    _sc_body,
    out_type=jax.ShapeDtypeStruct((NW, BPW, DP), jnp.float32),
    mesh=plsc.VectorSubcoreMesh(core_axis_name="c", subcore_axis_name="s",
                                num_cores=NC, num_subcores=NS),
    scratch_types=[
        pltpu.VMEM((L // 8, 8, BPW), jnp.int32),
        pltpu.VMEM((2, CH, BPW, DP), jnp.float32),
        pltpu.VMEM((BPW, DP), jnp.float32),
        pltpu.VMEM((2 + HID, DP), jnp.float32),
        pltpu.SemaphoreType.DMA,
        pltpu.SemaphoreType.DMA,
    ],
    compiler_params=pltpu.CompilerParams(use_tc_tiling_on_sc=False),
)


@jax.jit
def kernel(inputs, embed_weight, W1, b1, W2, b2):
    idxt = (inputs.astype(jnp.int32).T
            .reshape(L // 8, 8, NW, BPW).transpose(0, 2, 1, 3))
    embt = embed_weight.T
    w1p = jnp.zeros((D, DP), jnp.float32).at[:, :HID].set(W1.T)
    proj = _project(embt, w1p)

    consts = jnp.zeros((2 + HID, DP), jnp.float32)
    consts = consts.at[0, :HID].set(b1)
    consts = consts.at[1, :NCLS].set(b2)
    consts = consts.at[2:2 + HID, :NCLS].set(W2.T)

    out = _sc_call(proj.reshape(VOCAB, DP), idxt, consts)
    return out.reshape(B, DP)[:, :NCLS]
